# Initial kernel scaffold; baseline (speedup 1.0000x reference)
import jax
import jax.numpy as jnp
from jax.experimental import pallas as pl


def kernel(text, text_lengths, emb, W1, b1, W2, b2):
    raise NotImplementedError("write your pallas kernel here")


# speedup vs baseline: 17.8287x; 17.8287x over previous
import functools

import jax
import jax.numpy as jnp
from jax import lax
from jax.experimental import pallas as pl
from jax.experimental.pallas import tpu as pltpu
from jax.experimental.pallas import tpu_sc as plsc

_NC = 2
_NS = 16
_NW = _NC * _NS
_LANE = 16


def _make_pool(B, L, H, CH):
    rows_per_w = B // _NW
    n_chunks = rows_per_w // CH
    n_vreg = H // _LANE
    inv_l = 1.0 / L

    mesh = plsc.VectorSubcoreMesh(
        core_axis_name="c", subcore_axis_name="s",
        num_cores=_NC, num_subcores=_NS)

    @functools.partial(
        pl.kernel,
        out_type=jax.ShapeDtypeStruct((B, H), jnp.float32),
        mesh=mesh,
        scratch_types=[
            pltpu.VMEM((CH * L,), jnp.int32),
            pltpu.VMEM((2, L, H), jnp.float32),
            pltpu.VMEM((CH, H), jnp.float32),
            pltpu.SemaphoreType.DMA,
            pltpu.SemaphoreType.DMA,
        ],
    )
    def pool(text_hbm, emb_hbm, out_hbm, idx_v, rows_v, ostage_v, sem0, sem1):
        wid = lax.axis_index("s") * _NC + lax.axis_index("c")
        base = wid * rows_per_w
        sems = (sem0, sem1)

        def start(r, slot):
            pltpu.async_copy(emb_hbm.at[idx_v.at[pl.ds(r * L, L)]],
                             rows_v.at[slot], sems[slot])

        def finish(r, slot):
            pltpu.make_async_copy(emb_hbm.at[idx_v.at[pl.ds(r * L, L)]],
                                  rows_v.at[slot], sems[slot]).wait()

        def reduce_row(slot, r_out):
            def body(t, acc):
                return tuple(acc[j] + rows_v[slot, t, pl.ds(j * _LANE, _LANE)]
                             for j in range(n_vreg))
            acc = lax.fori_loop(
                0, L, body,
                tuple(jnp.zeros((_LANE,), jnp.float32)
                      for _ in range(n_vreg)))
            for j in range(n_vreg):
                ostage_v[r_out, pl.ds(j * _LANE, _LANE)] = acc[j] * inv_l

        def chunk_body(c, carry):
            row0 = base + c * CH
            pltpu.sync_copy(text_hbm.at[pl.ds(row0 * L, CH * L)], idx_v)
            start(0, 0)

            def pair_body(p, carry2):
                r0 = 2 * p
                r1 = r0 + 1
                start(r1, 1)
                finish(r0, 0)
                reduce_row(0, r0)

                @pl.when(r0 + 2 < CH)
                def _():
                    start(r0 + 2, 0)

                finish(r1, 1)
                reduce_row(1, r1)
                return carry2

            lax.fori_loop(0, CH // 2, pair_body, 0)
            pltpu.sync_copy(ostage_v, out_hbm.at[pl.ds(row0, CH)])
            return carry

        lax.fori_loop(0, n_chunks, chunk_body, 0)

    return pool


def _make_mlp(B, H, F1, F2, BLK):
    def body(x_ref, w1_ref, b1_ref, w2_ref, b2_ref, o_ref):
        x = x_ref[...]
        h = jnp.dot(x, w1_ref[...], preferred_element_type=jnp.float32)
        h = jnp.maximum(h + b1_ref[...], 0.0)
        o = jnp.dot(h, w2_ref[...], preferred_element_type=jnp.float32)
        o_ref[...] = o + b2_ref[...]

    return pl.pallas_call(
        body,
        grid=(B // BLK,),
        in_specs=[
            pl.BlockSpec((BLK, H), lambda i: (i, 0)),
            pl.BlockSpec((H, F1), lambda i: (0, 0)),
            pl.BlockSpec((1, F1), lambda i: (0, 0)),
            pl.BlockSpec((F1, F2), lambda i: (0, 0)),
            pl.BlockSpec((1, F2), lambda i: (0, 0)),
        ],
        out_specs=pl.BlockSpec((BLK, F2), lambda i: (i, 0)),
        out_shape=jax.ShapeDtypeStruct((B, F2), jnp.float32),
    )


def kernel(text, text_lengths, emb, W1, b1, W2, b2):
    del text_lengths
    B, L = text.shape
    H = emb.shape[1]
    F1 = W1.shape[1]
    F2 = W2.shape[1]
    text = text.astype(jnp.int32).reshape(B * L)
    pooled = _make_pool(B, L, H, CH=64)(text, emb)
    mlp = _make_mlp(B, H, F1, F2, BLK=2048)
    return mlp(pooled, W1, b1.reshape(1, F1), W2, b2.reshape(1, F2))


# speedup vs baseline: 17.8391x; 1.0006x over previous
import functools

import jax
import jax.numpy as jnp
from jax import lax
from jax.experimental import pallas as pl
from jax.experimental.pallas import tpu as pltpu
from jax.experimental.pallas import tpu_sc as plsc

_NC = 2
_NS = 16
_NW = _NC * _NS
_LANE = 16


def _make_pool(B, L, H, CH):
    rows_per_w = B // _NW
    n_chunks = rows_per_w // CH
    n_vreg = H // _LANE
    inv_l = 1.0 / L

    mesh = plsc.VectorSubcoreMesh(
        core_axis_name="c", subcore_axis_name="s",
        num_cores=_NC, num_subcores=_NS)

    @functools.partial(
        pl.kernel,
        out_type=jax.ShapeDtypeStruct((B, H), jnp.float32),
        mesh=mesh,
        scratch_types=[
            pltpu.VMEM((CH * L,), jnp.int32),
            pltpu.VMEM((2, L, H), jnp.float32),
            pltpu.VMEM((CH, H), jnp.float32),
            pltpu.SemaphoreType.DMA,
            pltpu.SemaphoreType.DMA,
        ],
    )
    def pool(text_hbm, emb_hbm, out_hbm, idx_v, rows_v, ostage_v, sem0, sem1):
        wid = lax.axis_index("s") * _NC + lax.axis_index("c")
        base = wid * rows_per_w
        sems = (sem0, sem1)

        def start(r, slot):
            pltpu.async_copy(emb_hbm.at[idx_v.at[pl.ds(r * L, L)]],
                             rows_v.at[slot], sems[slot])

        def finish(r, slot):
            pltpu.make_async_copy(emb_hbm.at[idx_v.at[pl.ds(r * L, L)]],
                                  rows_v.at[slot], sems[slot]).wait()

        def reduce_row(slot, r_out):
            def body(t, acc):
                return tuple(acc[j] + rows_v[slot, t, pl.ds(j * _LANE, _LANE)]
                             for j in range(n_vreg))
            acc = lax.fori_loop(
                0, L, body,
                tuple(jnp.zeros((_LANE,), jnp.float32)
                      for _ in range(n_vreg)),
                unroll=8)
            for j in range(n_vreg):
                ostage_v[r_out, pl.ds(j * _LANE, _LANE)] = acc[j] * inv_l

        def chunk_body(c, carry):
            row0 = base + c * CH
            pltpu.sync_copy(text_hbm.at[pl.ds(row0 * L, CH * L)], idx_v)
            start(0, 0)

            def pair_body(p, carry2):
                r0 = 2 * p
                r1 = r0 + 1
                start(r1, 1)
                finish(r0, 0)
                reduce_row(0, r0)

                @pl.when(r0 + 2 < CH)
                def _():
                    start(r0 + 2, 0)

                finish(r1, 1)
                reduce_row(1, r1)
                return carry2

            lax.fori_loop(0, CH // 2, pair_body, 0)
            pltpu.sync_copy(ostage_v, out_hbm.at[pl.ds(row0, CH)])
            return carry

        lax.fori_loop(0, n_chunks, chunk_body, 0)

    return pool


def _make_mlp(B, H, F1, F2, BLK):
    def body(x_ref, w1_ref, b1_ref, w2_ref, b2_ref, o_ref):
        x = x_ref[...]
        h = jnp.dot(x, w1_ref[...], preferred_element_type=jnp.float32)
        h = jnp.maximum(h + b1_ref[...], 0.0)
        o = jnp.dot(h, w2_ref[...], preferred_element_type=jnp.float32)
        o_ref[...] = o + b2_ref[...]

    return pl.pallas_call(
        body,
        grid=(B // BLK,),
        in_specs=[
            pl.BlockSpec((BLK, H), lambda i: (i, 0)),
            pl.BlockSpec((H, F1), lambda i: (0, 0)),
            pl.BlockSpec((1, F1), lambda i: (0, 0)),
            pl.BlockSpec((F1, F2), lambda i: (0, 0)),
            pl.BlockSpec((1, F2), lambda i: (0, 0)),
        ],
        out_specs=pl.BlockSpec((BLK, F2), lambda i: (i, 0)),
        out_shape=jax.ShapeDtypeStruct((B, F2), jnp.float32),
    )


def kernel(text, text_lengths, emb, W1, b1, W2, b2):
    del text_lengths
    B, L = text.shape
    H = emb.shape[1]
    F1 = W1.shape[1]
    F2 = W2.shape[1]
    text = text.astype(jnp.int32).reshape(B * L)
    pooled = _make_pool(B, L, H, CH=64)(text, emb)
    mlp = _make_mlp(B, H, F1, F2, BLK=2048)
    return mlp(pooled, W1, b1.reshape(1, F1), W2, b2.reshape(1, F2))


# speedup vs baseline: 20.8356x; 1.1680x over previous
import functools

import jax
import jax.numpy as jnp
from jax import lax
from jax.experimental import pallas as pl
from jax.experimental.pallas import tpu as pltpu
from jax.experimental.pallas import tpu_sc as plsc

_NC = 2
_NS = 16
_NW = _NC * _NS
_LANE = 16


def _make_pool(B, L, H, CH):
    rows_per_w = B // _NW
    n_chunks = rows_per_w // CH
    n_vreg = H // _LANE
    inv_l = 1.0 / L

    mesh = plsc.VectorSubcoreMesh(
        core_axis_name="c", subcore_axis_name="s",
        num_cores=_NC, num_subcores=_NS)

    @functools.partial(
        pl.kernel,
        out_type=jax.ShapeDtypeStruct((B, H), jnp.float32),
        mesh=mesh,
        scratch_types=[
            pltpu.VMEM((CH * L,), jnp.int32),
            pltpu.VMEM((4, L, H), jnp.float32),
            pltpu.VMEM((CH, H), jnp.float32),
            pltpu.SemaphoreType.DMA,
            pltpu.SemaphoreType.DMA,
            pltpu.SemaphoreType.DMA,
            pltpu.SemaphoreType.DMA,
        ],
    )
    def pool(text_hbm, emb_hbm, out_hbm, idx_v, rows_v, ostage_v,
             sem0, sem1, sem2, sem3):
        wid = lax.axis_index("s") * _NC + lax.axis_index("c")
        base = wid * rows_per_w
        sems = (sem0, sem1, sem2, sem3)

        def start(r, slot):
            pltpu.async_copy(emb_hbm.at[idx_v.at[pl.ds(r * L, L)]],
                             rows_v.at[slot], sems[slot])

        def finish(r, slot):
            pltpu.make_async_copy(emb_hbm.at[idx_v.at[pl.ds(r * L, L)]],
                                  rows_v.at[slot], sems[slot]).wait()

        def reduce_row(slot, r_out):
            def body(t, acc):
                return tuple(acc[j] + rows_v[slot, t, pl.ds(j * _LANE, _LANE)]
                             for j in range(n_vreg))
            acc = lax.fori_loop(
                0, L, body,
                tuple(jnp.zeros((_LANE,), jnp.float32)
                      for _ in range(n_vreg)),
                unroll=8)
            for j in range(n_vreg):
                ostage_v[r_out, pl.ds(j * _LANE, _LANE)] = acc[j] * inv_l

        def chunk_body(c, carry):
            row0 = base + c * CH
            pltpu.sync_copy(text_hbm.at[pl.ds(row0 * L, CH * L)], idx_v)
            for k in range(3):
                start(k, k)

            def quad_body(q, carry2):
                r0 = 4 * q
                for k in range(4):
                    r = r0 + k

                    @pl.when(r + 3 < CH)
                    def _(r=r, k=k):
                        start(r + 3, (k + 3) % 4)

                    finish(r, k)
                    reduce_row(k, r)
                return carry2

            lax.fori_loop(0, CH // 4, quad_body, 0)
            pltpu.sync_copy(ostage_v, out_hbm.at[pl.ds(row0, CH)])
            return carry

        lax.fori_loop(0, n_chunks, chunk_body, 0)

    return pool


def _make_mlp(B, H, F1, F2, BLK):
    def body(x_ref, w1_ref, b1_ref, w2_ref, b2_ref, o_ref):
        x = x_ref[...]
        h = jnp.dot(x, w1_ref[...], preferred_element_type=jnp.float32)
        h = jnp.maximum(h + b1_ref[...], 0.0)
        o = jnp.dot(h, w2_ref[...], preferred_element_type=jnp.float32)
        o_ref[...] = o + b2_ref[...]

    return pl.pallas_call(
        body,
        grid=(B // BLK,),
        in_specs=[
            pl.BlockSpec((BLK, H), lambda i: (i, 0)),
            pl.BlockSpec((H, F1), lambda i: (0, 0)),
            pl.BlockSpec((1, F1), lambda i: (0, 0)),
            pl.BlockSpec((F1, F2), lambda i: (0, 0)),
            pl.BlockSpec((1, F2), lambda i: (0, 0)),
        ],
        out_specs=pl.BlockSpec((BLK, F2), lambda i: (i, 0)),
        out_shape=jax.ShapeDtypeStruct((B, F2), jnp.float32),
    )


def kernel(text, text_lengths, emb, W1, b1, W2, b2):
    del text_lengths
    B, L = text.shape
    H = emb.shape[1]
    F1 = W1.shape[1]
    F2 = W2.shape[1]
    text = text.astype(jnp.int32).reshape(B * L)
    pooled = _make_pool(B, L, H, CH=32)(text, emb)
    mlp = _make_mlp(B, H, F1, F2, BLK=2048)
    return mlp(pooled, W1, b1.reshape(1, F1), W2, b2.reshape(1, F2))


# speedup vs baseline: 21.8301x; 1.0477x over previous
import functools

import jax
import jax.numpy as jnp
from jax import lax
from jax.experimental import pallas as pl
from jax.experimental.pallas import tpu as pltpu
from jax.experimental.pallas import tpu_sc as plsc

_NC = 2
_NS = 16
_NW = _NC * _NS
_LANE = 16


def _make_pool(B, L, H, CH):
    rows_per_w = B // _NW
    n_chunks = rows_per_w // CH
    n_vreg = H // _LANE
    inv_l = 1.0 / L

    mesh = plsc.VectorSubcoreMesh(
        core_axis_name="c", subcore_axis_name="s",
        num_cores=_NC, num_subcores=_NS)

    @functools.partial(
        pl.kernel,
        out_type=jax.ShapeDtypeStruct((B, H), jnp.float32),
        mesh=mesh,
        scratch_types=[
            pltpu.VMEM((CH * L,), jnp.int32),
            pltpu.VMEM((4, L, H), jnp.float32),
            pltpu.VMEM((CH, H), jnp.float32),
            pltpu.SemaphoreType.DMA,
            pltpu.SemaphoreType.DMA,
            pltpu.SemaphoreType.DMA,
            pltpu.SemaphoreType.DMA,
        ],
    )
    def pool(text_hbm, emb_hbm, out_hbm, idx_v, rows_v, ostage_v,
             sem0, sem1, sem2, sem3):
        wid = lax.axis_index("s") * _NC + lax.axis_index("c")
        base = wid * rows_per_w
        sems = (sem0, sem1, sem2, sem3)

        def start(r, slot):
            pltpu.async_copy(emb_hbm.at[idx_v.at[pl.ds(r * L, L)]],
                             rows_v.at[slot], sems[slot])

        def finish(r, slot):
            pltpu.make_async_copy(emb_hbm.at[idx_v.at[pl.ds(r * L, L)]],
                                  rows_v.at[slot], sems[slot]).wait()

        def reduce_row(slot, r_out):
            def body(t, acc):
                return tuple(acc[j] + rows_v[slot, t, pl.ds(j * _LANE, _LANE)]
                             for j in range(n_vreg))
            acc = lax.fori_loop(
                0, L, body,
                tuple(jnp.zeros((_LANE,), jnp.float32)
                      for _ in range(n_vreg)),
                unroll=8)
            for j in range(n_vreg):
                ostage_v[r_out, pl.ds(j * _LANE, _LANE)] = acc[j] * inv_l

        def chunk_body(c, carry):
            row0 = base + c * CH
            pltpu.sync_copy(text_hbm.at[pl.ds(row0 * L, CH * L)], idx_v)
            for k in range(3):
                start(k, k)

            def quad_body(q, carry2):
                r0 = 4 * q
                for k in range(4):
                    r = r0 + k

                    @pl.when(r + 3 < CH)
                    def _(r=r, k=k):
                        start(r + 3, (k + 3) % 4)

                    finish(r, k)
                    reduce_row(k, r)
                return carry2

            lax.fori_loop(0, CH // 4, quad_body, 0)
            pltpu.sync_copy(ostage_v, out_hbm.at[pl.ds(row0, CH)])
            return carry

        lax.fori_loop(0, n_chunks, chunk_body, 0)

    return pool


def _make_mlp(B, H, F1, F2, BLK):
    def body(x_ref, w1_ref, b1_ref, w2_ref, b2_ref, o_ref):
        x = x_ref[...]
        h = jnp.dot(x, w1_ref[...], preferred_element_type=jnp.float32)
        h = jnp.maximum(h + b1_ref[...], 0.0)
        o = jnp.dot(h, w2_ref[...], preferred_element_type=jnp.float32)
        o_ref[...] = o + b2_ref[...]

    return pl.pallas_call(
        body,
        grid=(B // BLK,),
        in_specs=[
            pl.BlockSpec((BLK, H), lambda i: (i, 0)),
            pl.BlockSpec((H, F1), lambda i: (0, 0)),
            pl.BlockSpec((1, F1), lambda i: (0, 0)),
            pl.BlockSpec((F1, F2), lambda i: (0, 0)),
            pl.BlockSpec((1, F2), lambda i: (0, 0)),
        ],
        out_specs=pl.BlockSpec((BLK, F2), lambda i: (i, 0)),
        out_shape=jax.ShapeDtypeStruct((B, F2), jnp.float32),
    )


def kernel(text, text_lengths, emb, W1, b1, W2, b2):
    del text_lengths
    B, L = text.shape
    H = emb.shape[1]
    F1 = W1.shape[1]
    F2 = W2.shape[1]
    text = text.astype(jnp.int32).reshape(B * L)
    pooled = _make_pool(B, L, H, CH=64)(text, emb)
    mlp = _make_mlp(B, H, F1, F2, BLK=2048)
    return mlp(pooled, W1, b1.reshape(1, F1), W2, b2.reshape(1, F2))
